# baseline (speedup 1.0000x reference)
import functools

import jax
import jax.numpy as jnp
from jax import lax
from jax.experimental import pallas as pl
from jax.experimental.pallas import tpu as pltpu
from jax.experimental.pallas import tpu_sc as plsc

HIDDEN = 64
ROW2 = 2 * HIDDEN
BATCH = 4096
HIST = 200
VOCAB2 = 500000
B_TOTAL = BATCH * HIST
NC, NS = 2, 16
NW = NC * NS
G = 128
CB = 256
GPC = CB // G
TPH = BATCH // CB
NTILE = HIST * TPH
TPW = NTILE // NW
NPAIR = TPW // 2
L = 16


def _emb_body(x_hbm, table_hbm, out_hbm,
              idx_v0, idx_v1, rows_v0, rows_v1,
              outt_v0, outt_v1, g0, g1, s0, s1):
    wid = lax.axis_index("s") * NC + lax.axis_index("c")
    t0 = wid * TPW
    lane_iota = lax.iota(jnp.int32, L)
    idx_v = (idx_v0, idx_v1)
    rows_v = (rows_v0, rows_v1)
    outt_v = (outt_v0, outt_v1)
    gsem = (g0, g1)
    ssem = (s0, s1)

    def stage_idx(c, b):
        pltpu.sync_copy(x_hbm.at[pl.ds((t0 + c) * CB, CB)], idx_v[b])

    def fire_gathers(b):
        for j in range(GPC):
            pltpu.async_copy(
                table_hbm.at[idx_v[b].at[pl.ds(j * G, G)]],
                rows_v[b].at[pl.ds(j * G, G)],
                gsem[b],
            )

    def wait_gathers(b):
        for j in range(GPC):
            pltpu.make_async_copy(
                table_hbm.at[idx_v[b].at[pl.ds(j * G, G)]],
                rows_v[b].at[pl.ds(j * G, G)],
                gsem[b],
            ).wait()

    def store_descr(c, b):
        t = t0 + c
        h = t // TPH
        b0 = (t % TPH) * CB
        return pltpu.make_async_copy(
            outt_v[b], out_hbm.at[h, :, pl.ds(b0, CB)], ssem[b])

    fvecs = [k * L + lane_iota for k in range(HIDDEN // L)]

    def transpose(b):
        @plsc.parallel_loop(0, CB // L)
        def _tr(g):
            for cc in range(L):
                c = g * L + cc
                cvec = jnp.full((L,), c, jnp.int32)
                for k in range(HIDDEN // L):
                    vals = rows_v[b][c, pl.ds(k * L, L)]
                    plsc.store_scatter(outt_v[b], [fvecs[k], cvec], vals)

    for b in range(2):
        stage_idx(b, b)
        fire_gathers(b)

    def pair_body(p, carry):
        for b in range(2):
            c = 2 * p + b
            wait_gathers(b)

            @pl.when(p > 0)
            def _():
                store_descr(c - 2, b).wait()

            transpose(b)
            store_descr(c, b).start()

            @pl.when(p < NPAIR - 1)
            def _():
                stage_idx(c + 2, b)
                fire_gathers(b)
        return carry

    lax.fori_loop(0, NPAIR, pair_body, 0)

    for b in range(2):
        store_descr(TPW - 2 + b, b).wait()


@jax.jit
def _embed(x_flat, table):
    mesh = plsc.VectorSubcoreMesh(core_axis_name="c", subcore_axis_name="s")
    k = pl.kernel(
        _emb_body,
        out_type=jax.ShapeDtypeStruct((HIST, HIDDEN, BATCH), jnp.float32),
        mesh=mesh,
        compiler_params=pltpu.CompilerParams(use_tc_tiling_on_sc=False,
                                             needs_layout_passes=False),
        scratch_types=[
            pltpu.VMEM((CB,), jnp.int32),
            pltpu.VMEM((CB,), jnp.int32),
            pltpu.VMEM((CB, HIDDEN), jnp.float32),
            pltpu.VMEM((CB, HIDDEN), jnp.float32),
            pltpu.VMEM((HIDDEN, CB), jnp.float32),
            pltpu.VMEM((HIDDEN, CB), jnp.float32),
            pltpu.SemaphoreType.DMA,
            pltpu.SemaphoreType.DMA,
            pltpu.SemaphoreType.DMA,
            pltpu.SemaphoreType.DMA,
        ],
    )
    return k(x_flat, table)


def kernel(x, table):
    b, h = x.shape
    x_flat = x.T.reshape(B_TOTAL)
    out3 = _embed(x_flat, table)
    return out3.transpose(2, 0, 1)


# speedup vs baseline: 1.2707x; 1.2707x over previous
import functools

import jax
import jax.numpy as jnp
from jax import lax
from jax.experimental import pallas as pl
from jax.experimental.pallas import tpu as pltpu
from jax.experimental.pallas import tpu_sc as plsc

HIDDEN = 64
B_TOTAL = 4096 * 200
NC, NS = 2, 16
NW = NC * NS
BPW = B_TOTAL // NW
G = 128
CH = 512
GPC = CH // G
NCHUNK = BPW // CH
NPAIR = NCHUNK // 2


def _emb_body(x_hbm, table_hbm, out_hbm,
              idx_v0, idx_v1, rows0, rows1, g0, g1, s0, s1):
    wid = lax.axis_index("s") * NC + lax.axis_index("c")
    base = wid * BPW
    idx_v = (idx_v0, idx_v1)
    rows = (rows0, rows1)
    gsem = (g0, g1)
    ssem = (s0, s1)

    def stage_idx(c, b):
        pltpu.sync_copy(x_hbm.at[pl.ds(base + c * CH, CH)], idx_v[b])

    def fire_gathers(b):
        for j in range(GPC):
            pltpu.async_copy(
                table_hbm.at[idx_v[b].at[pl.ds(j * G, G)]],
                rows[b].at[pl.ds(j * G, G)],
                gsem[b],
            )

    def wait_gathers(b):
        for j in range(GPC):
            pltpu.make_async_copy(
                table_hbm.at[idx_v[b].at[pl.ds(j * G, G)]],
                rows[b].at[pl.ds(j * G, G)],
                gsem[b],
            ).wait()

    def store_descr(c, b):
        off = base + c * CH
        return pltpu.make_async_copy(rows[b], out_hbm.at[pl.ds(off, CH)],
                                     ssem[b])

    for b in range(2):
        stage_idx(b, b)
        fire_gathers(b)

    def pair_body(p, carry):
        for b in range(2):
            c = 2 * p + b
            wait_gathers(b)
            store_descr(c, b).start()

            @pl.when(p < NPAIR - 1)
            def _():
                store_descr(c, b).wait()
                stage_idx(c + 2, b)
                fire_gathers(b)
        return carry

    lax.fori_loop(0, NPAIR, pair_body, 0)

    for b in range(2):
        store_descr(NCHUNK - 2 + b, b).wait()


@jax.jit
def _embed(x_flat, table):
    mesh = plsc.VectorSubcoreMesh(core_axis_name="c", subcore_axis_name="s")
    k = pl.kernel(
        _emb_body,
        out_type=jax.ShapeDtypeStruct((B_TOTAL, HIDDEN), jnp.float32),
        mesh=mesh,
        compiler_params=pltpu.CompilerParams(use_tc_tiling_on_sc=False),
        scratch_types=[
            pltpu.VMEM((CH,), jnp.int32),
            pltpu.VMEM((CH,), jnp.int32),
            pltpu.VMEM((CH, HIDDEN), jnp.float32),
            pltpu.VMEM((CH, HIDDEN), jnp.float32),
            pltpu.SemaphoreType.DMA,
            pltpu.SemaphoreType.DMA,
            pltpu.SemaphoreType.DMA,
            pltpu.SemaphoreType.DMA,
        ],
    )
    return k(x_flat, table)


def kernel(x, table):
    b, h = x.shape
    x_flat = x.reshape(B_TOTAL)
    out = _embed(x_flat, table)
    return out.reshape(b, h, HIDDEN)
